# baseline (speedup 1.0000x reference)
import functools

import jax
import jax.numpy as jnp
from jax import lax
from jax.experimental import pallas as pl
from jax.experimental.pallas import tpu as pltpu
from jax.experimental.pallas import tpu_sc as plsc

THETA_F = 1.15
FILTER_F = 1e-08
NEG_BIG = -1e30

NC = 2
NS = 16
NW = NC * NS
LANES = 16
CHUNK = 128


def _tile_flatten(depth, pred):
    B, C, H, W = depth.shape
    RG = 8
    RPG = H // RG
    out_rows = B * C * H * W // 128

    def body(d_ref, p_ref, do_ref, po_ref):
        for ref, oref in ((d_ref, do_ref), (p_ref, po_ref)):
            xx = ref[0, 0]
            i = 0
            for k in range(RPG // 8):
                for c in range(W // 128):
                    oref[pl.ds(i * 8, 8), :] = xx[8 * k:8 * k + 8,
                                                  128 * c:128 * c + 128]
                    i += 1

    return pl.pallas_call(
        body,
        grid=(B, RG),
        in_specs=[
            pl.BlockSpec((1, 1, RPG, W), lambda b, j: (b, 0, j, 0)),
            pl.BlockSpec((1, 1, RPG, W), lambda b, j: (b, 0, j, 0)),
        ],
        out_specs=[
            pl.BlockSpec((RPG * W // 128, 128), lambda b, j: (b * RG + j, 0)),
            pl.BlockSpec((RPG * W // 128, 128), lambda b, j: (b * RG + j, 0)),
        ],
        out_shape=[
            jax.ShapeDtypeStruct((out_rows, 128), jnp.float32),
            jax.ShapeDtypeStruct((out_rows, 128), jnp.float32),
        ],
    )(depth, pred)


def _sc_gather_logits(n_per_w):
    mesh = plsc.VectorSubcoreMesh(core_axis_name="c", subcore_axis_name="s")
    n_chunks = n_per_w // CHUNK

    @functools.partial(
        pl.kernel,
        mesh=mesh,
        out_type=jax.ShapeDtypeStruct((NW * n_per_w,), jnp.float32),
        scratch_types=[
            pltpu.VMEM((n_per_w,), jnp.int32),
            pltpu.VMEM((n_per_w,), jnp.int32),
            pltpu.VMEM((n_per_w,), jnp.float32),
            pltpu.VMEM((n_per_w,), jnp.float32),
            pltpu.VMEM((n_per_w,), jnp.float32),
            pltpu.VMEM((n_per_w,), jnp.float32),
            pltpu.VMEM((n_per_w,), jnp.float32),
            pltpu.SemaphoreType.DMA,
        ],
    )
    def sc_kernel(d_hbm, p_hbm, ia_hbm, ib_hbm, x_hbm,
                  ia_v, ib_v, za_v, zb_v, pa_v, pb_v, x_v, sem):
        wid = lax.axis_index("s") * NC + lax.axis_index("c")
        base = wid * n_per_w
        pltpu.sync_copy(ia_hbm.at[pl.ds(base, n_per_w)], ia_v)
        pltpu.sync_copy(ib_hbm.at[pl.ds(base, n_per_w)], ib_v)
        handles = []
        for j in range(n_chunks):
            for iv in (ia_v, ib_v):
                for k in range(CHUNK // LANES):
                    sl = pl.ds(j * CHUNK + k * LANES, LANES)
                    f = iv[sl]
                    q = ((f & jnp.int32(-4096))
                         | (((f >> 7) & jnp.int32(3)) << 10)
                         | (((f >> 9) & jnp.int32(7)) << 7)
                         | (f & jnp.int32(127)))
                    iv[sl] = q
            s = pl.ds(j * CHUNK, CHUNK)
            handles.append(pltpu.async_copy(d_hbm.at[ia_v.at[s]], za_v.at[s], sem))
            handles.append(pltpu.async_copy(d_hbm.at[ib_v.at[s]], zb_v.at[s], sem))
            handles.append(pltpu.async_copy(p_hbm.at[ia_v.at[s]], pa_v.at[s], sem))
            handles.append(pltpu.async_copy(p_hbm.at[ib_v.at[s]], pb_v.at[s], sem))
        for h in handles:
            h.wait()
        one = jnp.float32(1.0)
        neg_one = jnp.float32(-1.0)
        zero = jnp.float32(0.0)
        for i in range(n_per_w // LANES):
            s = pl.ds(i * LANES, LANES)
            za = za_v[s]
            zb = zb_v[s]
            pa = pa_v[s]
            pb = pb_v[s]
            keep = (za > FILTER_F) | (zb > FILTER_F)
            t = jnp.where(za / zb > THETA_F, neg_one,
                          jnp.where(zb / za > THETA_F, one, zero))
            valid = keep & (t != zero)
            x = jnp.where(valid, -t * (pa - pb), jnp.float32(NEG_BIG))
            x_v[s] = x
        pltpu.sync_copy(x_v, x_hbm.at[pl.ds(base, n_per_w)])

    return sc_kernel


def _softplus_sum(x_ref, o_ref):
    x = x_ref[...]
    o_ref[...] = jnp.sum(jnp.log1p(jnp.exp(x))).reshape(1, 1)


def kernel(depth, pred, idx_A, idx_B):
    n = idx_A.shape[0]
    n_per_w = n // NW
    d_lin, p_lin = _tile_flatten(depth, pred)
    d_flat = d_lin.reshape(-1)
    p_flat = p_lin.reshape(-1)
    x = _sc_gather_logits(n_per_w)(d_flat, p_flat, idx_A, idx_B)
    loss = pl.pallas_call(
        _softplus_sum,
        out_shape=jax.ShapeDtypeStruct((1, 1), jnp.float32),
    )(x.reshape(n // CHUNK, CHUNK))
    return loss[0, 0]


# speedup vs baseline: 2.0619x; 2.0619x over previous
import functools

import jax
import jax.numpy as jnp
from jax import lax
from jax.experimental import pallas as pl
from jax.experimental.pallas import tpu as pltpu
from jax.experimental.pallas import tpu_sc as plsc

THETA_F = 1.15
FILTER_F = 1e-08
NEG_BIG = -1e30

NC = 2
NS = 16
NW = NC * NS
LANES = 16
CHUNK = 128


def _tile_flatten(depth, pred):
    B, C, H, W = depth.shape
    out_rows = B * C * H * W // 128
    rows_per_img = H * W // 128

    def body(d_ref, p_ref, do_ref, po_ref):
        for ref, oref in ((d_ref, do_ref), (p_ref, po_ref)):
            xx = ref[0, 0]
            i = 0
            for k in range(H // 8):
                for c in range(W // 128):
                    oref[pl.ds(i * 8, 8), :] = xx[8 * k:8 * k + 8,
                                                  128 * c:128 * c + 128]
                    i += 1

    return pl.pallas_call(
        body,
        grid=(B,),
        in_specs=[
            pl.BlockSpec((1, 1, H, W), lambda b: (b, 0, 0, 0)),
            pl.BlockSpec((1, 1, H, W), lambda b: (b, 0, 0, 0)),
        ],
        out_specs=[
            pl.BlockSpec((rows_per_img, 128), lambda b: (b, 0)),
            pl.BlockSpec((rows_per_img, 128), lambda b: (b, 0)),
        ],
        out_shape=[
            jax.ShapeDtypeStruct((out_rows, 128), jnp.float32),
            jax.ShapeDtypeStruct((out_rows, 128), jnp.float32),
        ],
    )(depth, pred)


def _sc_gather_logits(n_per_w):
    mesh = plsc.VectorSubcoreMesh(core_axis_name="c", subcore_axis_name="s")
    n_chunks = n_per_w // CHUNK

    @functools.partial(
        pl.kernel,
        mesh=mesh,
        out_type=jax.ShapeDtypeStruct((NW * n_per_w,), jnp.float32),
        scratch_types=[
            pltpu.VMEM((n_per_w,), jnp.int32),
            pltpu.VMEM((n_per_w,), jnp.int32),
            pltpu.VMEM((n_per_w,), jnp.float32),
            pltpu.VMEM((n_per_w,), jnp.float32),
            pltpu.VMEM((n_per_w,), jnp.float32),
            pltpu.VMEM((n_per_w,), jnp.float32),
            pltpu.VMEM((n_per_w,), jnp.float32),
            pltpu.SemaphoreType.DMA,
        ],
    )
    def sc_kernel(d_hbm, p_hbm, ia_hbm, ib_hbm, x_hbm,
                  ia_v, ib_v, za_v, zb_v, pa_v, pb_v, x_v, sem):
        wid = lax.axis_index("s") * NC + lax.axis_index("c")
        base = wid * n_per_w
        pltpu.sync_copy(ia_hbm.at[pl.ds(base, n_per_w)], ia_v)
        pltpu.sync_copy(ib_hbm.at[pl.ds(base, n_per_w)], ib_v)
        handles = []
        for j in range(n_chunks):
            for iv in (ia_v, ib_v):
                for k in range(CHUNK // LANES):
                    sl = pl.ds(j * CHUNK + k * LANES, LANES)
                    f = iv[sl]
                    q = ((f & jnp.int32(-4096))
                         | (((f >> 7) & jnp.int32(3)) << 10)
                         | (((f >> 9) & jnp.int32(7)) << 7)
                         | (f & jnp.int32(127)))
                    iv[sl] = q
            s = pl.ds(j * CHUNK, CHUNK)
            handles.append(pltpu.async_copy(d_hbm.at[ia_v.at[s]], za_v.at[s], sem))
            handles.append(pltpu.async_copy(d_hbm.at[ib_v.at[s]], zb_v.at[s], sem))
            handles.append(pltpu.async_copy(p_hbm.at[ia_v.at[s]], pa_v.at[s], sem))
            handles.append(pltpu.async_copy(p_hbm.at[ib_v.at[s]], pb_v.at[s], sem))
        for h in handles:
            h.wait()
        one = jnp.float32(1.0)
        neg_one = jnp.float32(-1.0)
        zero = jnp.float32(0.0)
        for i in range(n_per_w // LANES):
            s = pl.ds(i * LANES, LANES)
            za = za_v[s]
            zb = zb_v[s]
            pa = pa_v[s]
            pb = pb_v[s]
            keep = (za > FILTER_F) | (zb > FILTER_F)
            t = jnp.where(za / zb > THETA_F, neg_one,
                          jnp.where(zb / za > THETA_F, one, zero))
            valid = keep & (t != zero)
            x = jnp.where(valid, -t * (pa - pb), jnp.float32(NEG_BIG))
            x_v[s] = x
        pltpu.sync_copy(x_v, x_hbm.at[pl.ds(base, n_per_w)])

    return sc_kernel


def _softplus_sum(x_ref, o_ref):
    x = x_ref[...]
    o_ref[...] = jnp.sum(jnp.log1p(jnp.exp(x))).reshape(1, 1)


def kernel(depth, pred, idx_A, idx_B):
    n = idx_A.shape[0]
    n_per_w = n // NW
    d_lin, p_lin = _tile_flatten(depth, pred)
    d_flat = d_lin.reshape(-1)
    p_flat = p_lin.reshape(-1)
    x = _sc_gather_logits(n_per_w)(d_flat, p_flat, idx_A, idx_B)
    loss = pl.pallas_call(
        _softplus_sum,
        out_shape=jax.ShapeDtypeStruct((1, 1), jnp.float32),
    )(x.reshape(n // CHUNK, CHUNK))
    return loss[0, 0]
